# baseline (speedup 1.0000x reference)
import functools

import jax
import jax.numpy as jnp
from jax import lax
from jax.experimental import pallas as pl
from jax.experimental.pallas import tpu as pltpu
from jax.experimental.pallas import tpu_sc as plsc

_NC = 2
_NS = 16
_NW = _NC * _NS
_L = 16
_CHUNK = 128


def kernel(text_indices, context_indices, w_in, w_out):
    B = text_indices.shape[0]
    N, E = w_in.shape
    assert B % (_NW * _L) == 0 and E % _L == 0
    b_per_w = B // _NW
    n_chunks = b_per_w // _CHUNK
    mesh = plsc.VectorSubcoreMesh(core_axis_name="c", subcore_axis_name="s")

    @functools.partial(
        pl.kernel,
        mesh=mesh,
        out_type=jax.ShapeDtypeStruct((B,), jnp.float32),
        compiler_params=pltpu.CompilerParams(
            needs_layout_passes=False, use_tc_tiling_on_sc=False),
        scratch_types=[
            pltpu.VMEM((b_per_w,), jnp.int32),
            pltpu.VMEM((b_per_w,), jnp.int32),
            pltpu.VMEM((b_per_w, E), jnp.float32),
            pltpu.VMEM((b_per_w, E), jnp.float32),
            pltpu.VMEM((b_per_w,), jnp.float32),
            pltpu.SemaphoreType.DMA,
        ],
    )
    def sc_kernel(ti_hbm, ci_hbm, win_hbm, wout_hbm, out_hbm,
                  ti_v, ci_v, x_v, c_v, o_v, sem):
        wid = lax.axis_index("s") * _NC + lax.axis_index("c")
        base = wid * b_per_w
        pltpu.sync_copy(ti_hbm.at[pl.ds(base, b_per_w)], ti_v)
        pltpu.sync_copy(ci_hbm.at[pl.ds(base, b_per_w)], ci_v)
        copies = []
        for j in range(n_chunks):
            sl = pl.ds(j * _CHUNK, _CHUNK)
            copies.append(pltpu.async_copy(
                win_hbm.at[ti_v.at[sl]], x_v.at[sl], sem))
            copies.append(pltpu.async_copy(
                wout_hbm.at[ci_v.at[sl]], c_v.at[sl], sem))
        for cp in copies:
            cp.wait()

        lane = lax.iota(jnp.int32, _L)

        def body(g, carry):
            bvec = g * _L + lane
            acc = jnp.zeros((_L,), jnp.float32)
            for d in range(E):
                dvec = jnp.full((_L,), d, jnp.int32)
                xv = plsc.load_gather(x_v, [bvec, dvec])
                cv = plsc.load_gather(c_v, [bvec, dvec])
                acc = acc + xv * cv
            o_v[pl.ds(g * _L, _L)] = acc
            return carry

        lax.fori_loop(0, b_per_w // _L, body, 0)
        pltpu.sync_copy(o_v, out_hbm.at[pl.ds(base, b_per_w)])

    return sc_kernel(text_indices, context_indices, w_in, w_out)


# speedup vs baseline: 1.9981x; 1.9981x over previous
import functools

import jax
import jax.numpy as jnp
from jax import lax
from jax.experimental import pallas as pl
from jax.experimental.pallas import tpu as pltpu
from jax.experimental.pallas import tpu_sc as plsc

_NC = 2
_NS = 16
_L = 16
_CK = 4096
_DH = 4
_H0 = 50048
_HM = 49920


def kernel(text_indices, context_indices, w_in, w_out):
    B = text_indices.shape[0]
    N, E = w_in.shape
    assert E == _NC * _NS and B % _CK == 0
    tail = N - _H0 - _HM
    assert 0 < tail <= _L * 2
    b_per_t = B // _NS
    mesh = plsc.VectorSubcoreMesh(core_axis_name="c", subcore_axis_name="s")

    @functools.partial(
        pl.kernel,
        mesh=mesh,
        out_type=jax.ShapeDtypeStruct((_NC, B), jnp.float32),
        compiler_params=pltpu.CompilerParams(needs_layout_passes=False),
        scratch_types=[
            pltpu.VMEM((_H0,), jnp.float32),
            pltpu.VMEM((tail,), jnp.float32),
            pltpu.VMEM((B,), jnp.int32),
            pltpu.VMEM((B,), jnp.float32),
            pltpu.VMEM((B,), jnp.float32),
            pltpu.VMEM((_CK,), jnp.float32),
            pltpu.VMEM((_DH, B // _NS), jnp.float32),
            pltpu.VMEM((B // _NS,), jnp.float32),
            pltpu.VMEM_SHARED((_NS, B), jnp.float32),
        ],
    )
    def sc_kernel(ti_hbm, ci_hbm, wt_in_hbm, wt_out_hbm, out_hbm,
                  row_v, tail_v, idx_v, xfull, cfull, gout_v, pbuf, obuf,
                  p_sp):
        c = lax.axis_index("c")
        t = lax.axis_index("s")
        d = c * _NS + t

        for tbl_hbm, i_hbm, dst in (
            (wt_in_hbm, ti_hbm, xfull),
            (wt_out_hbm, ci_hbm, cfull),
        ):
            pltpu.sync_copy(i_hbm, idx_v)
            for h in range(2):
                if h == 0:
                    pltpu.sync_copy(tbl_hbm.at[d, pl.ds(0, _H0)], row_v)
                else:
                    pltpu.sync_copy(tbl_hbm.at[d, pl.ds(_H0, _HM)],
                                    row_v.at[pl.ds(0, _HM)])
                    pltpu.sync_copy(tbl_hbm.at[d, pl.ds(_H0 + _HM, tail)],
                                    tail_v)

                def gbody(g, carry):
                    base = g * (4 * _L)
                    for u in range(4):
                        sl = pl.ds(base + u * _L, _L)
                        iv = idx_v[sl]
                        if h == 0:
                            m = iv < _H0
                            gv = plsc.load_gather(row_v, [iv], mask=m)
                            dst[sl] = gv
                        else:
                            m1 = (iv >= _H0) & (iv < _H0 + _HM)
                            g1 = plsc.load_gather(row_v, [iv - _H0], mask=m1)
                            m2 = iv >= _H0 + _HM
                            g2 = plsc.load_gather(
                                tail_v, [iv - (_H0 + _HM)], mask=m2)
                            dst[sl] = jnp.where(
                                m1, g1, jnp.where(m2, g2, dst[sl]))
                    return carry

                lax.fori_loop(0, B // (4 * _L), gbody, 0)

        for chunk in range(B // _CK):
            cbase = chunk * _CK

            def pbody(g, carry):
                base = g * (4 * _L)
                for u in range(4):
                    o = base + u * _L
                    gout_v[pl.ds(o, _L)] = (
                        xfull[pl.ds(cbase + o, _L)]
                        * cfull[pl.ds(cbase + o, _L)])
                return carry

            lax.fori_loop(0, _CK // (4 * _L), pbody, 0)
            pltpu.sync_copy(gout_v, p_sp.at[t, pl.ds(cbase, _CK)])

        plsc.subcore_barrier()

        bbase = t * b_per_t
        for dchunk in range(_NS // _DH):
            dbase = dchunk * _DH
            pltpu.sync_copy(p_sp.at[pl.ds(dbase, _DH), pl.ds(bbase, b_per_t)],
                            pbuf)

            def rbody(v, carry):
                sl = pl.ds(v * _L, _L)
                acc = pbuf[0, sl]
                for dd in range(1, _DH):
                    acc = acc + pbuf[dd, sl]
                if dchunk:
                    acc = acc + obuf[sl]
                obuf[sl] = acc
                return carry

            lax.fori_loop(0, b_per_t // _L, rbody, 0)
        pltpu.sync_copy(obuf, out_hbm.at[c, pl.ds(bbase, b_per_t)])

    partials = sc_kernel(text_indices, context_indices, w_in.T, w_out.T)
    return partials[0] + partials[1]
